# baseline (speedup 1.0000x reference)
import jax
import jax.numpy as jnp
import numpy as np
from jax.experimental import pallas as pl
from jax.experimental.pallas import tpu as pltpu

_DIM = 384
_QK = 384
_NWIN = 7
_HEADS = 8
_TOPK = 4
_RATIO = 4
_WS = 32
_P2 = _NWIN * _NWIN
_HW = _WS * _WS
_CH = _QK // _HEADS
_W2D = (_WS // _RATIO) ** 2
_CKV = _QK + _DIM
_KSEL = _TOPK * _W2D
_SCALE = _QK ** -0.5
_IMG = _NWIN * _WS


def _pool_matrix() -> np.ndarray:
    p = np.zeros((_W2D, _HW), np.float32)
    for o in range(_W2D):
        r0, c0 = divmod(o, _WS // _RATIO)
        for dr in range(_RATIO):
            for dc in range(_RATIO):
                p[o, (r0 * _RATIO + dr) * _WS + (c0 * _RATIO + dc)] = 1.0 / 16.0
    return p


_POOL = _pool_matrix()


def _qkv_kernel(x_ref, wbf_ref, wr_ref, b_ref, pool_ref,
                q_ref, v_ref, kvp_ref, qw_ref, kw_ref):
    xw = x_ref[...].reshape(_HW, _DIM)
    qkv = jnp.dot(xw.astype(jnp.bfloat16), wbf_ref[...],
                  preferred_element_type=jnp.float32) + b_ref[...]
    q = qkv[:, :_QK]
    kv = qkv[:, _QK:]
    q_ref[0] = q.astype(jnp.bfloat16)
    v_ref[...] = kv[:, _QK:].reshape(_WS, _WS, _DIM)
    kvp_ref[0] = jnp.dot(pool_ref[...], kv,
                         preferred_element_type=jnp.float32).astype(jnp.bfloat16)
    xbar = jnp.mean(xw, axis=0, keepdims=True)
    qkw = jnp.dot(xbar, wr_ref[...], preferred_element_type=jnp.float32)
    qw_ref[0, 0] = qkw[0, :_QK] + b_ref[0, :_QK]
    kw_ref[0, 0] = qkw[0, _QK:] + b_ref[0, _QK:2 * _QK]


def _router_kernel(qw_ref, kw_ref, idx_ref):
    logits = jax.lax.dot_general(
        qw_ref[...].reshape(_P2, _QK) * _SCALE, kw_ref[...].reshape(_P2, _QK),
        (((1,), (1,)), ((), ())), preferred_element_type=jnp.float32)
    col = jax.lax.broadcasted_iota(jnp.int32, (_P2, _P2), 1)
    picks = []
    for _ in range(_TOPK):
        m = jnp.max(logits, axis=1, keepdims=True)
        idx = jnp.min(jnp.where(logits >= m, col, jnp.int32(1 << 30)), axis=1)
        picks.append(idx)
        logits = jnp.where(col == idx[:, None], -jnp.inf, logits)
    idx_ref[...] = jnp.stack(picks, axis=1)


def _attn_kernel(idx_ref, q_ref, kvp_ref, o_ref):
    p = pl.program_id(0)
    q = q_ref[0]
    parts = [kvp_ref[pl.ds(idx_ref[p, t], 1)] for t in range(_TOPK)]
    kv_sel = jnp.concatenate(parts, axis=0).reshape(_KSEL, _CKV)
    k_sel = kv_sel[:, :_QK]
    v_sel = kv_sel[:, _QK:]
    outs = []
    for h in range(_HEADS):
        sl = slice(h * _CH, (h + 1) * _CH)
        logits = jax.lax.dot_general(
            q[:, sl], k_sel[:, sl], (((1,), (1,)), ((), ())),
            preferred_element_type=jnp.float32) * _SCALE
        e = jnp.exp(logits)
        rden = 1.0 / jnp.sum(e, axis=1, keepdims=True)
        av = jnp.dot(e.astype(jnp.bfloat16), v_sel[:, sl],
                     preferred_element_type=jnp.float32)
        outs.append(av * rden)
    out = jnp.concatenate(outs, axis=1)
    o_ref[...] = out.reshape(_WS, _WS, _DIM)


_STRIP = 16
_NSTRIP = _IMG // _STRIP


def _lepe_kernel(v_ref, vt_ref, vb_ref, a_ref, lw_ref, lb_ref, wo_ref, bo_ref,
                 o_ref):
    s = pl.program_id(0)
    v = v_ref[...]
    top = jnp.where(s == 0, 0.0, vt_ref[...])
    bot = jnp.where(s == _NSTRIP - 1, 0.0, vb_ref[...])
    vp = jnp.concatenate([top, v, bot], axis=0)
    zc = jnp.zeros((_STRIP + 2, 1, _DIM), jnp.float32)
    vp = jnp.concatenate([zc, vp, zc], axis=1)
    shifts = [vp[:, dc:dc + _IMG, :] for dc in range(3)]
    acc = jnp.zeros((_STRIP, _IMG, _DIM), jnp.float32) + lb_ref[...].reshape(1, 1, _DIM)
    for dr in range(3):
        for dc in range(3):
            acc = acc + shifts[dc][dr:dr + _STRIP] * lw_ref[dr * 3 + dc]
    y = (a_ref[...] + acc).reshape(_STRIP * _IMG, _DIM)
    out = jnp.dot(y.astype(jnp.bfloat16), wo_ref[...],
                  preferred_element_type=jnp.float32) + bo_ref[...]
    o_ref[...] = out.reshape(_STRIP, _IMG, _DIM)


@jax.jit
def kernel(x, W_qkv, b_qkv, lepe_w, lepe_b, Wo, bo):
    f32 = jnp.float32
    bf16 = jnp.bfloat16
    x_img = jnp.transpose(x[0], (1, 2, 0))

    q, v_img, kv_pix, q_win, k_win = pl.pallas_call(
        _qkv_kernel,
        grid=(_NWIN, _NWIN),
        in_specs=[
            pl.BlockSpec((_WS, _WS, _DIM), lambda jw, iw: (jw, iw, 0)),
            pl.BlockSpec((_DIM, 2 * _QK + _DIM), lambda jw, iw: (0, 0)),
            pl.BlockSpec((_DIM, 2 * _QK), lambda jw, iw: (0, 0)),
            pl.BlockSpec((1, 2 * _QK + _DIM), lambda jw, iw: (0, 0)),
            pl.BlockSpec((_W2D, _HW), lambda jw, iw: (0, 0)),
        ],
        out_specs=[
            pl.BlockSpec((1, _HW, _QK), lambda jw, iw: (jw * _NWIN + iw, 0, 0)),
            pl.BlockSpec((_WS, _WS, _DIM), lambda jw, iw: (jw, iw, 0)),
            pl.BlockSpec((1, _W2D, _CKV), lambda jw, iw: (jw * _NWIN + iw, 0, 0)),
            pl.BlockSpec((1, 1, _QK), lambda jw, iw: (jw * _NWIN + iw, 0, 0)),
            pl.BlockSpec((1, 1, _QK), lambda jw, iw: (jw * _NWIN + iw, 0, 0)),
        ],
        out_shape=[
            jax.ShapeDtypeStruct((_P2, _HW, _QK), bf16),
            jax.ShapeDtypeStruct((_IMG, _IMG, _DIM), f32),
            jax.ShapeDtypeStruct((_P2, _W2D, _CKV), bf16),
            jax.ShapeDtypeStruct((_P2, 1, _QK), f32),
            jax.ShapeDtypeStruct((_P2, 1, _QK), f32),
        ],
    )(x_img, W_qkv.astype(bf16), W_qkv[:, :2 * _QK], b_qkv.reshape(1, -1),
      jnp.asarray(_POOL))

    topk_index = pl.pallas_call(
        _router_kernel,
        out_shape=jax.ShapeDtypeStruct((_P2, _TOPK), jnp.int32),
    )(q_win, k_win)

    attn_img = pl.pallas_call(
        _attn_kernel,
        grid_spec=pltpu.PrefetchScalarGridSpec(
            num_scalar_prefetch=1,
            grid=(_P2,),
            in_specs=[
                pl.BlockSpec((1, _HW, _QK), lambda p, idx: (p, 0, 0)),
                pl.BlockSpec((_P2, _W2D, _CKV), lambda p, idx: (0, 0, 0)),
            ],
            out_specs=pl.BlockSpec(
                (_WS, _WS, _DIM), lambda p, idx: (p // _NWIN, p % _NWIN, 0)),
        ),
        out_shape=jax.ShapeDtypeStruct((_IMG, _IMG, _DIM), f32),
    )(topk_index, q, kv_pix)

    lw = jnp.transpose(lepe_w[:, 0], (1, 2, 0)).reshape(9, _DIM)
    out_img = pl.pallas_call(
        _lepe_kernel,
        grid=(_NSTRIP,),
        in_specs=[
            pl.BlockSpec((_STRIP, _IMG, _DIM), lambda s: (s, 0, 0)),
            pl.BlockSpec((1, _IMG, _DIM),
                         lambda s: (jnp.maximum(s * _STRIP - 1, 0), 0, 0)),
            pl.BlockSpec((1, _IMG, _DIM),
                         lambda s: (jnp.minimum(s * _STRIP + _STRIP, _IMG - 1), 0, 0)),
            pl.BlockSpec((_STRIP, _IMG, _DIM), lambda s: (s, 0, 0)),
            pl.BlockSpec((9, _DIM), lambda s: (0, 0)),
            pl.BlockSpec((1, _DIM), lambda s: (0, 0)),
            pl.BlockSpec((_DIM, _DIM), lambda s: (0, 0)),
            pl.BlockSpec((1, _DIM), lambda s: (0, 0)),
        ],
        out_specs=pl.BlockSpec((_STRIP, _IMG, _DIM), lambda s: (s, 0, 0)),
        out_shape=jax.ShapeDtypeStruct((_IMG, _IMG, _DIM), f32),
    )(v_img, v_img, v_img, attn_img, lw, lepe_b.reshape(1, -1),
      Wo.astype(bf16), bo.reshape(1, -1))

    return jnp.transpose(out_img, (2, 0, 1))[None]


# speedup vs baseline: 1.2937x; 1.2937x over previous
import jax
import jax.numpy as jnp
import numpy as np
from jax.experimental import pallas as pl
from jax.experimental.pallas import tpu as pltpu

_DIM = 384
_QK = 384
_NWIN = 7
_HEADS = 8
_TOPK = 4
_RATIO = 4
_WS = 32
_P2 = _NWIN * _NWIN
_HW = _WS * _WS
_CH = _QK // _HEADS
_W2D = (_WS // _RATIO) ** 2
_CKV = _QK + _DIM
_KSEL = _TOPK * _W2D
_SCALE = _QK ** -0.5
_IMG = _NWIN * _WS


def _pool_matrix() -> np.ndarray:
    p = np.zeros((_W2D, _HW), np.float32)
    for o in range(_W2D):
        r0, c0 = divmod(o, _WS // _RATIO)
        for dr in range(_RATIO):
            for dc in range(_RATIO):
                p[o, (r0 * _RATIO + dr) * _WS + (c0 * _RATIO + dc)] = 1.0 / 16.0
    return p


_POOL = _pool_matrix()


def _qkv_kernel(x_ref, w_ref, b_ref, pool_ref,
                q_ref, v_ref, kvp_ref, qw_ref, kw_ref):
    xw = x_ref[...].reshape(_HW, _DIM)
    qkv = jnp.dot(xw, w_ref[...], preferred_element_type=jnp.float32) + b_ref[...]
    q = qkv[:, :_QK]
    kv = qkv[:, _QK:]
    q_ref[0] = q
    v_ref[...] = kv[:, _QK:].reshape(_WS, _WS, _DIM)
    kvp_ref[0] = jnp.dot(pool_ref[...], kv, preferred_element_type=jnp.float32)
    qw_ref[0, 0] = jnp.mean(q, axis=0)
    kw_ref[0, 0] = jnp.mean(kv[:, :_QK], axis=0)


def _router_kernel(qw_ref, kw_ref, idx_ref):
    logits = jax.lax.dot_general(
        qw_ref[...].reshape(_P2, _QK) * _SCALE, kw_ref[...].reshape(_P2, _QK),
        (((1,), (1,)), ((), ())), preferred_element_type=jnp.float32)
    col = jax.lax.broadcasted_iota(jnp.int32, (_P2, _P2), 1)
    picks = []
    for _ in range(_TOPK):
        m = jnp.max(logits, axis=1, keepdims=True)
        idx = jnp.min(jnp.where(logits >= m, col, jnp.int32(1 << 30)), axis=1)
        picks.append(idx)
        logits = jnp.where(col == idx[:, None], -jnp.inf, logits)
    idx_ref[...] = jnp.stack(picks, axis=1)


def _attn_kernel(idx_ref, q_ref, kvp_ref, o_ref):
    p = pl.program_id(0)
    q = q_ref[0]
    parts = [kvp_ref[pl.ds(idx_ref[p, t], 1)] for t in range(_TOPK)]
    kv_sel = jnp.concatenate(parts, axis=0).reshape(_KSEL, _CKV)
    k_sel = kv_sel[:, :_QK]
    v_sel = kv_sel[:, _QK:]
    outs = []
    for h in range(_HEADS):
        sl = slice(h * _CH, (h + 1) * _CH)
        logits = jax.lax.dot_general(
            q[:, sl], k_sel[:, sl], (((1,), (1,)), ((), ())),
            preferred_element_type=jnp.float32) * _SCALE
        e = jnp.exp(logits)
        rden = 1.0 / jnp.sum(e, axis=1, keepdims=True)
        av = jnp.dot(e, v_sel[:, sl],
                     preferred_element_type=jnp.float32)
        outs.append(av * rden)
    out = jnp.concatenate(outs, axis=1)
    o_ref[...] = out.reshape(_WS, _WS, _DIM)


_STRIP = 16
_NSTRIP = _IMG // _STRIP


def _lepe_kernel(v_ref, vt_ref, vb_ref, a_ref, lw_ref, lb_ref, wo_ref, bo_ref,
                 o_ref):
    s = pl.program_id(0)
    v = v_ref[...]
    top = jnp.where(s == 0, 0.0, vt_ref[...])
    bot = jnp.where(s == _NSTRIP - 1, 0.0, vb_ref[...])
    vp = jnp.concatenate([top, v, bot], axis=0)
    zc = jnp.zeros((_STRIP + 2, 1, _DIM), jnp.float32)
    vp = jnp.concatenate([zc, vp, zc], axis=1)
    shifts = [vp[:, dc:dc + _IMG, :] for dc in range(3)]
    acc = jnp.zeros((_STRIP, _IMG, _DIM), jnp.float32) + lb_ref[...].reshape(1, 1, _DIM)
    for dr in range(3):
        for dc in range(3):
            acc = acc + shifts[dc][dr:dr + _STRIP] * lw_ref[dr * 3 + dc]
    y = (a_ref[...] + acc).reshape(_STRIP * _IMG, _DIM)
    out = jnp.dot(y, wo_ref[...], preferred_element_type=jnp.float32) + bo_ref[...]
    o_ref[...] = out.reshape(_STRIP, _IMG, _DIM)


@jax.jit
def kernel(x, W_qkv, b_qkv, lepe_w, lepe_b, Wo, bo):
    f32 = jnp.float32
    x_img = jnp.transpose(x[0], (1, 2, 0))

    q, v_img, kv_pix, q_win, k_win = pl.pallas_call(
        _qkv_kernel,
        grid=(_NWIN, _NWIN),
        in_specs=[
            pl.BlockSpec((_WS, _WS, _DIM), lambda jw, iw: (jw, iw, 0)),
            pl.BlockSpec((_DIM, 2 * _QK + _DIM), lambda jw, iw: (0, 0)),
            pl.BlockSpec((1, 2 * _QK + _DIM), lambda jw, iw: (0, 0)),
            pl.BlockSpec((_W2D, _HW), lambda jw, iw: (0, 0)),
        ],
        out_specs=[
            pl.BlockSpec((1, _HW, _QK), lambda jw, iw: (jw * _NWIN + iw, 0, 0)),
            pl.BlockSpec((_WS, _WS, _DIM), lambda jw, iw: (jw, iw, 0)),
            pl.BlockSpec((1, _W2D, _CKV), lambda jw, iw: (jw * _NWIN + iw, 0, 0)),
            pl.BlockSpec((1, 1, _QK), lambda jw, iw: (jw * _NWIN + iw, 0, 0)),
            pl.BlockSpec((1, 1, _QK), lambda jw, iw: (jw * _NWIN + iw, 0, 0)),
        ],
        out_shape=[
            jax.ShapeDtypeStruct((_P2, _HW, _QK), f32),
            jax.ShapeDtypeStruct((_IMG, _IMG, _DIM), f32),
            jax.ShapeDtypeStruct((_P2, _W2D, _CKV), f32),
            jax.ShapeDtypeStruct((_P2, 1, _QK), f32),
            jax.ShapeDtypeStruct((_P2, 1, _QK), f32),
        ],
    )(x_img, W_qkv, b_qkv.reshape(1, -1), jnp.asarray(_POOL))

    topk_index = pl.pallas_call(
        _router_kernel,
        out_shape=jax.ShapeDtypeStruct((_P2, _TOPK), jnp.int32),
    )(q_win, k_win)

    attn_img = pl.pallas_call(
        _attn_kernel,
        grid_spec=pltpu.PrefetchScalarGridSpec(
            num_scalar_prefetch=1,
            grid=(_P2,),
            in_specs=[
                pl.BlockSpec((1, _HW, _QK), lambda p, idx: (p, 0, 0)),
                pl.BlockSpec((_P2, _W2D, _CKV), lambda p, idx: (0, 0, 0)),
            ],
            out_specs=pl.BlockSpec(
                (_WS, _WS, _DIM), lambda p, idx: (p // _NWIN, p % _NWIN, 0)),
        ),
        out_shape=jax.ShapeDtypeStruct((_IMG, _IMG, _DIM), f32),
    )(topk_index, q, kv_pix)

    lw = jnp.transpose(lepe_w[:, 0], (1, 2, 0)).reshape(9, _DIM)
    out_img = pl.pallas_call(
        _lepe_kernel,
        grid=(_NSTRIP,),
        in_specs=[
            pl.BlockSpec((_STRIP, _IMG, _DIM), lambda s: (s, 0, 0)),
            pl.BlockSpec((1, _IMG, _DIM),
                         lambda s: (jnp.maximum(s * _STRIP - 1, 0), 0, 0)),
            pl.BlockSpec((1, _IMG, _DIM),
                         lambda s: (jnp.minimum(s * _STRIP + _STRIP, _IMG - 1), 0, 0)),
            pl.BlockSpec((_STRIP, _IMG, _DIM), lambda s: (s, 0, 0)),
            pl.BlockSpec((9, _DIM), lambda s: (0, 0)),
            pl.BlockSpec((1, _DIM), lambda s: (0, 0)),
            pl.BlockSpec((_DIM, _DIM), lambda s: (0, 0)),
            pl.BlockSpec((1, _DIM), lambda s: (0, 0)),
        ],
        out_specs=pl.BlockSpec((_STRIP, _IMG, _DIM), lambda s: (s, 0, 0)),
        out_shape=jax.ShapeDtypeStruct((_IMG, _IMG, _DIM), f32),
    )(v_img, v_img, v_img, attn_img, lw, lepe_b.reshape(1, -1), Wo,
      bo.reshape(1, -1))

    return jnp.transpose(out_img, (2, 0, 1))[None]


# speedup vs baseline: 2.0301x; 1.5693x over previous
import jax
import jax.numpy as jnp
import numpy as np
from jax.experimental import pallas as pl
from jax.experimental.pallas import tpu as pltpu

_DIM = 384
_QK = 384
_NWIN = 7
_HEADS = 8
_TOPK = 4
_RATIO = 4
_WS = 32
_P2 = _NWIN * _NWIN
_HW = _WS * _WS
_CH = _QK // _HEADS
_W2D = (_WS // _RATIO) ** 2
_CKV = _QK + _DIM
_KSEL = _TOPK * _W2D
_SCALE = _QK ** -0.5
_IMG = _NWIN * _WS


def _pool_matrix() -> np.ndarray:
    p = np.zeros((_W2D, _HW), np.float32)
    for o in range(_W2D):
        r0, c0 = divmod(o, _WS // _RATIO)
        for dr in range(_RATIO):
            for dc in range(_RATIO):
                p[o, (r0 * _RATIO + dr) * _WS + (c0 * _RATIO + dc)] = 1.0 / 16.0
    return p


_POOL = _pool_matrix()


def _qkv_kernel(x_ref, w_ref, b_ref, pool_ref,
                q_ref, v_ref, kvp_ref, qw_ref, kw_ref):
    xw = x_ref[...].reshape(_HW, _DIM)
    qkv = jnp.dot(xw, w_ref[...], preferred_element_type=jnp.float32) + b_ref[...]
    q = qkv[:, :_QK]
    kv = qkv[:, _QK:]
    qw_ref[0, 0] = jnp.mean(q, axis=0)
    kw_ref[0, 0] = jnp.mean(kv[:, :_QK], axis=0)
    q_ref[...] = q * _SCALE
    v_ref[...] = kv[:, _QK:].reshape(_WS, _WS, _DIM)
    kvp_ref[...] = jnp.dot(pool_ref[...], kv, preferred_element_type=jnp.float32)


def _router_kernel(qw_ref, kw_ref, idx_ref):
    logits = jax.lax.dot_general(
        qw_ref[...].reshape(_P2, _QK) * _SCALE, kw_ref[...].reshape(_P2, _QK),
        (((1,), (1,)), ((), ())), preferred_element_type=jnp.float32)
    col = jax.lax.broadcasted_iota(jnp.int32, (_P2, _P2), 1)
    picks = []
    for _ in range(_TOPK):
        m = jnp.max(logits, axis=1, keepdims=True)
        idx = jnp.min(jnp.where(logits >= m, col, jnp.int32(1 << 30)), axis=1)
        picks.append(idx)
        logits = jnp.where(col == idx[:, None], -jnp.inf, logits)
    idx_ref[...] = jnp.stack(picks, axis=1)


def _attn_kernel(idx_ref, q_ref, kvp_ref, o_ref):
    p = pl.program_id(0)
    q = q_ref[...]
    parts = [kvp_ref[pl.ds(idx_ref[p, t] * _W2D, _W2D)] for t in range(_TOPK)]
    kv_sel = jnp.concatenate(parts, axis=0)
    k_sel = kv_sel[:, :_QK]
    v_sel = kv_sel[:, _QK:]
    sls = [slice(h * _CH, (h + 1) * _CH) for h in range(_HEADS)]
    es = [jnp.exp(jax.lax.dot_general(
        q[:, sl], k_sel[:, sl], (((1,), (1,)), ((), ())),
        preferred_element_type=jnp.float32)) for sl in sls]
    rdens = [1.0 / jnp.sum(e, axis=1, keepdims=True) for e in es]
    avs = [jnp.dot(e, v_sel[:, sl], preferred_element_type=jnp.float32)
           for e, sl in zip(es, sls)]
    out = jnp.concatenate([av * rden for av, rden in zip(avs, rdens)], axis=1)
    o_ref[...] = out.reshape(_WS, _WS, _DIM)


_STRIP = 16
_NSTRIP = _IMG // _STRIP


def _lepe_kernel(v_ref, vt_ref, vb_ref, a_ref, lw_ref, lb_ref, wo_ref, bo_ref,
                 o_ref):
    s = pl.program_id(0)
    v = v_ref[...]
    top = jnp.where(s == 0, 0.0, vt_ref[...])
    bot = jnp.where(s == _NSTRIP - 1, 0.0, vb_ref[...])
    vp = jnp.concatenate([top, v, bot], axis=0)
    zc = jnp.zeros((_STRIP + 2, 1, _DIM), jnp.float32)
    vp = jnp.concatenate([zc, vp, zc], axis=1)
    shifts = [vp[:, dc:dc + _IMG, :] for dc in range(3)]
    acc = jnp.zeros((_STRIP, _IMG, _DIM), jnp.float32) + lb_ref[...].reshape(1, 1, _DIM)
    for dr in range(3):
        for dc in range(3):
            acc = acc + shifts[dc][dr:dr + _STRIP] * lw_ref[dr * 3 + dc]
    y = (a_ref[...] + acc).reshape(_STRIP * _IMG, _DIM)
    out = jnp.dot(y, wo_ref[...], preferred_element_type=jnp.float32) + bo_ref[...]
    o_ref[...] = out.reshape(_STRIP, _IMG, _DIM)


@jax.jit
def kernel(x, W_qkv, b_qkv, lepe_w, lepe_b, Wo, bo):
    f32 = jnp.float32
    x_img = jnp.transpose(x[0], (1, 2, 0))

    q, v_img, kv_pix, q_win, k_win = pl.pallas_call(
        _qkv_kernel,
        grid=(_NWIN, _NWIN),
        in_specs=[
            pl.BlockSpec((_WS, _WS, _DIM), lambda jw, iw: (jw, iw, 0)),
            pl.BlockSpec((_DIM, 2 * _QK + _DIM), lambda jw, iw: (0, 0)),
            pl.BlockSpec((1, 2 * _QK + _DIM), lambda jw, iw: (0, 0)),
            pl.BlockSpec((_W2D, _HW), lambda jw, iw: (0, 0)),
        ],
        out_specs=[
            pl.BlockSpec((_HW, _QK), lambda jw, iw: (jw * _NWIN + iw, 0)),
            pl.BlockSpec((_WS, _WS, _DIM), lambda jw, iw: (jw, iw, 0)),
            pl.BlockSpec((_W2D, _CKV), lambda jw, iw: (jw * _NWIN + iw, 0)),
            pl.BlockSpec((1, 1, _QK), lambda jw, iw: (jw * _NWIN + iw, 0, 0)),
            pl.BlockSpec((1, 1, _QK), lambda jw, iw: (jw * _NWIN + iw, 0, 0)),
        ],
        out_shape=[
            jax.ShapeDtypeStruct((_P2 * _HW, _QK), f32),
            jax.ShapeDtypeStruct((_IMG, _IMG, _DIM), f32),
            jax.ShapeDtypeStruct((_P2 * _W2D, _CKV), f32),
            jax.ShapeDtypeStruct((_P2, 1, _QK), f32),
            jax.ShapeDtypeStruct((_P2, 1, _QK), f32),
        ],
    )(x_img, W_qkv, b_qkv.reshape(1, -1), jnp.asarray(_POOL))

    topk_index = pl.pallas_call(
        _router_kernel,
        out_shape=jax.ShapeDtypeStruct((_P2, _TOPK), jnp.int32),
    )(q_win, k_win)

    attn_img = pl.pallas_call(
        _attn_kernel,
        grid_spec=pltpu.PrefetchScalarGridSpec(
            num_scalar_prefetch=1,
            grid=(_P2,),
            in_specs=[
                pl.BlockSpec((_HW, _QK), lambda p, idx: (p, 0)),
                pl.BlockSpec((_P2 * _W2D, _CKV), lambda p, idx: (0, 0)),
            ],
            out_specs=pl.BlockSpec(
                (_WS, _WS, _DIM), lambda p, idx: (p // _NWIN, p % _NWIN, 0)),
        ),
        out_shape=jax.ShapeDtypeStruct((_IMG, _IMG, _DIM), f32),
    )(topk_index, q, kv_pix)

    lw = jnp.transpose(lepe_w[:, 0], (1, 2, 0)).reshape(9, _DIM)
    out_img = pl.pallas_call(
        _lepe_kernel,
        grid=(_NSTRIP,),
        in_specs=[
            pl.BlockSpec((_STRIP, _IMG, _DIM), lambda s: (s, 0, 0)),
            pl.BlockSpec((1, _IMG, _DIM),
                         lambda s: (jnp.maximum(s * _STRIP - 1, 0), 0, 0)),
            pl.BlockSpec((1, _IMG, _DIM),
                         lambda s: (jnp.minimum(s * _STRIP + _STRIP, _IMG - 1), 0, 0)),
            pl.BlockSpec((_STRIP, _IMG, _DIM), lambda s: (s, 0, 0)),
            pl.BlockSpec((9, _DIM), lambda s: (0, 0)),
            pl.BlockSpec((1, _DIM), lambda s: (0, 0)),
            pl.BlockSpec((_DIM, _DIM), lambda s: (0, 0)),
            pl.BlockSpec((1, _DIM), lambda s: (0, 0)),
        ],
        out_specs=pl.BlockSpec((_STRIP, _IMG, _DIM), lambda s: (s, 0, 0)),
        out_shape=jax.ShapeDtypeStruct((_IMG, _IMG, _DIM), f32),
    )(v_img, v_img, v_img, attn_img, lw, lepe_b.reshape(1, -1), Wo,
      bo.reshape(1, -1))

    return jnp.transpose(out_img, (2, 0, 1))[None]
